# baseline (speedup 1.0000x reference)
import functools

import jax
import jax.numpy as jnp
from jax import lax
from jax.experimental import pallas as pl
from jax.experimental.pallas import tpu as pltpu
from jax.experimental.pallas import tpu_sc as plsc

N = 20000
NUM_CLASSES = 21
SCORE_THRESHOLD = 0.5
IOU_THRESHOLD = 0.5

NC = 2
NS = 16
NROWS = NC * NS
L = 16
CHUNK = 128
SBUF = 2000
MAXK = N
NPAD = N + 8

_mesh = plsc.VectorSubcoreMesh(core_axis_name="c", subcore_axis_name="s")


@functools.partial(
    pl.kernel,
    out_type=jax.ShapeDtypeStruct((NPAD,), jnp.float32),
    mesh=_mesh,
    scratch_types=[
        pltpu.VMEM((SBUF,), jnp.int32),
        pltpu.VMEM((N + CHUNK,), jnp.int32),
        pltpu.VMEM((CHUNK,), jnp.int32),
        pltpu.VMEM((CHUNK,), jnp.float32),
        pltpu.VMEM((CHUNK,), jnp.float32),
        pltpu.VMEM((CHUNK,), jnp.float32),
        pltpu.VMEM((CHUNK,), jnp.float32),
        pltpu.VMEM((CHUNK,), jnp.float32),
        pltpu.VMEM((MAXK,), jnp.float32),
        pltpu.VMEM((MAXK,), jnp.float32),
        pltpu.VMEM((MAXK,), jnp.float32),
        pltpu.VMEM((MAXK,), jnp.float32),
        pltpu.SemaphoreType.DMA,
    ],
    compiler_params=pltpu.CompilerParams(needs_layout_passes=False),
)
def _nms_sc(bx1_h, by1_h, bx2_h, by2_h, cls_h, keep_h,
            cls_v, cand_v, idx_v, cx1_v, cy1_v, cx2_v, cy2_v, keep_v,
            kx1, ky1, kx2, ky2, sem):
    wid = lax.axis_index("s") * NC + lax.axis_index("c")
    zeros_l = jnp.zeros((L,), jnp.float32)
    lane0 = jnp.arange(L, dtype=jnp.int32) == 0
    ones_l = jnp.ones((L,), jnp.float32)
    iota_l = jnp.arange(L, dtype=jnp.int32)
    widv = jnp.full((L,), wid, jnp.int32)

    def stage_body(sb, ncand):
        pltpu.sync_copy(cls_h.at[pl.ds(sb * SBUF, SBUF)], cls_v)

        def compact_body(t, nc):
            chunk = cls_v[pl.ds(t * L, L)]
            mask = chunk == widv
            pos = iota_l + (sb * SBUF + t * L)
            dst = nc + plsc.cumsum(mask.astype(jnp.int32)) - 1
            plsc.store_scatter(cand_v, [dst], pos, mask=mask)
            return nc + plsc.all_reduce_population_count(mask)[0]

        return lax.fori_loop(0, SBUF // L, compact_body, ncand)

    cnt = lax.fori_loop(0, N // SBUF, stage_body, jnp.int32(0))
    nchunks = (cnt + (CHUNK - 1)) // CHUNK
    npadv = jnp.full((L,), N, jnp.int32)
    for t in range(CHUNK // L):
        cand_v[pl.ds(cnt + t * L, L)] = npadv

    def chunk_body(ci, kcount):
        base = ci * CHUNK
        for t in range(CHUNK // L):
            idx_v[pl.ds(t * L, L)] = cand_v[pl.ds(base + t * L, L)]
        c1 = pltpu.async_copy(bx1_h.at[idx_v], cx1_v, sem)
        c2 = pltpu.async_copy(by1_h.at[idx_v], cy1_v, sem)
        c3 = pltpu.async_copy(bx2_h.at[idx_v], cx2_v, sem)
        c4 = pltpu.async_copy(by2_h.at[idx_v], cy2_v, sem)
        c1.wait(); c2.wait(); c3.wait(); c4.wait()
        for t in range(CHUNK // L):
            keep_v[pl.ds(t * L, L)] = zeros_l
        nj = jnp.minimum(jnp.int32(CHUNK), cnt - base)

        def cand_body(j, k):
            jsplat = jnp.full((L,), j, jnp.int32)
            jx1 = plsc.load_gather(cx1_v, [jsplat])
            jy1 = plsc.load_gather(cy1_v, [jsplat])
            jx2 = plsc.load_gather(cx2_v, [jsplat])
            jy2 = plsc.load_gather(cy2_v, [jsplat])
            carea = (jx2 - jx1) * (jy2 - jy1)
            nkc = (k + (L - 1)) // L

            def scan_body(m, supp):
                off = m * L
                vx1 = kx1[pl.ds(off, L)]
                vy1 = ky1[pl.ds(off, L)]
                vx2 = kx2[pl.ds(off, L)]
                vy2 = ky2[pl.ds(off, L)]
                var = (vx2 - vx1) * (vy2 - vy1)
                ix1 = jnp.maximum(jx1, vx1)
                iy1 = jnp.maximum(jy1, vy1)
                ix2 = jnp.minimum(jx2, vx2)
                iy2 = jnp.minimum(jy2, vy2)
                inter = (jnp.maximum(ix2 - ix1, 0.0)
                         * jnp.maximum(iy2 - iy1, 0.0))
                union = (var + carea) - inter + jnp.float32(1e-9)
                hit = jnp.any(inter / union >= jnp.float32(IOU_THRESHOLD))
                return supp | hit

            supp = lax.fori_loop(0, nkc, scan_body, jnp.bool_(False))

            def do_keep(k):
                @pl.when(k % L == 0)
                def _():
                    kx1[pl.ds(k, L)] = zeros_l
                    ky1[pl.ds(k, L)] = zeros_l
                    kx2[pl.ds(k, L)] = zeros_l
                    ky2[pl.ds(k, L)] = zeros_l
                ksplat = jnp.full((L,), k, jnp.int32)
                plsc.store_scatter(kx1, [ksplat], jx1, mask=lane0)
                plsc.store_scatter(ky1, [ksplat], jy1, mask=lane0)
                plsc.store_scatter(kx2, [ksplat], jx2, mask=lane0)
                plsc.store_scatter(ky2, [ksplat], jy2, mask=lane0)
                plsc.store_scatter(keep_v, [jsplat], ones_l, mask=lane0)
                return k + 1

            return lax.cond(supp, lambda kk: kk, do_keep, k)

        kcount = lax.fori_loop(0, nj, cand_body, kcount)
        pltpu.async_copy(keep_v, keep_h.at[idx_v], sem).wait()
        return kcount

    lax.fori_loop(0, nchunks, chunk_body, jnp.int32(0))


def kernel(boxes, scores, labels):
    valid = scores >= SCORE_THRESHOLD
    order = jnp.argsort(jnp.where(valid, scores, jnp.float32(-1e30)))[::-1]
    b = boxes[order]
    s = scores[order]
    v = valid[order]
    cls = jnp.where(v, labels[order], NUM_CLASSES).astype(jnp.int32)

    pad = jnp.zeros((NPAD - N,), jnp.float32)
    keep_pad = _nms_sc(jnp.concatenate([b[:, 0], pad]),
                       jnp.concatenate([b[:, 1], pad]),
                       jnp.concatenate([b[:, 2], pad]),
                       jnp.concatenate([b[:, 3], pad]),
                       cls)

    keep = jnp.where(v, keep_pad[:N], jnp.float32(0.0))
    keep_f = keep[:, None]
    return jnp.concatenate([b * keep_f, s[:, None] * keep_f], axis=1)
